# baseline (speedup 1.0000x reference)
import functools

import jax
import jax.numpy as jnp
from jax import lax
from jax.experimental import pallas as pl
from jax.experimental.pallas import tpu as pltpu
from jax.experimental.pallas import tpu_sc as plsc

VOCAB = 1000000
D_MODEL = 64
SEQ_LEN = 200
BATCH = 1024
LANES = 16

NUM_CORES = 2
NUM_SUBCORES = 16
NW = NUM_CORES * NUM_SUBCORES

ROWS_TOTAL = BATCH * SEQ_LEN
ROWS_PER_W = ROWS_TOTAL // NW
CHUNK = 40
CHUNKS_PER_W = ROWS_PER_W // CHUNK
NBUF = SEQ_LEN // CHUNK
ROUNDS = CHUNKS_PER_W // NBUF
PAIR_ROWS = 2 * D_MODEL


def _body(x_hbm, table_hbm, pos_hbm, out_hbm,
          idx_v, pidx_v, pos_v, gbuf, wbuf, gsems, wsems):
    wid = lax.axis_index("s") * NUM_CORES + lax.axis_index("c")

    pltpu.sync_copy(pos_hbm, pos_v)
    pltpu.sync_copy(x_hbm.at[pl.ds(wid * ROWS_PER_W, ROWS_PER_W)], idx_v)

    def pre(i, carry):
        sl = pl.ds(i * LANES, LANES)
        pidx_v[sl] = lax.shift_right_logical(idx_v[sl], 1)
        return carry

    lax.fori_loop(0, ROWS_PER_W // LANES, pre, 0)

    def gather_start(c, b):
        pltpu.make_async_copy(
            table_hbm.at[pidx_v.at[pl.ds(c * CHUNK, CHUNK)]],
            gbuf.at[b], gsems.at[b]).start()

    def gather_wait(b):
        pltpu.make_async_copy(
            table_hbm.at[pl.ds(0, CHUNK)], gbuf.at[b], gsems.at[b]).wait()

    def write_start(c, b):
        pltpu.make_async_copy(
            wbuf.at[b], out_hbm.at[pl.ds(wid * ROWS_PER_W + c * CHUNK, CHUNK)],
            wsems.at[b]).start()

    def write_wait(c, b):
        pltpu.make_async_copy(
            wbuf.at[b], out_hbm.at[pl.ds(wid * ROWS_PER_W + c * CHUNK, CHUNK)],
            wsems.at[b]).wait()

    for b in range(NBUF):
        gather_start(b, b)

    lane_iota = lax.iota(jnp.int32, LANES)

    def round_body(r, carry):
        for b in range(NBUF):
            c = r * NBUF + b
            gather_wait(b)

            @pl.when(r > 0)
            def _():
                write_wait(c - NBUF, b)

            def add_row2(row, carry2):
                splat_row = lax.broadcast(row, (LANES,))
                tok = plsc.load_gather(
                    idx_v, [lax.broadcast(c * CHUNK + row, (LANES,))])
                hbase = lax.shift_left(
                    lax.bitwise_and(tok, jnp.int32(1)), jnp.int32(6))
                for j in range(D_MODEL // LANES):
                    lanes = hbase + lane_iota + jnp.int32(j * LANES)
                    vals = plsc.load_gather(
                        gbuf, [lax.broadcast(jnp.int32(b), (LANES,)),
                               splat_row, lanes])
                    pvals = plsc.load_gather(
                        pos_v, [lax.broadcast(b * CHUNK + row, (LANES,)),
                                lane_iota + jnp.int32(j * LANES)])
                    plsc.store_scatter(
                        wbuf, [lax.broadcast(jnp.int32(b), (LANES,)),
                               splat_row,
                               lane_iota + jnp.int32(j * LANES)],
                        vals + pvals)
                return carry2

            lax.fori_loop(0, CHUNK, add_row2, 0)

            @pl.when(r < ROUNDS - 1)
            def _():
                gather_start(c + NBUF, b)

            write_start(c, b)
        return carry

    lax.fori_loop(0, ROUNDS, round_body, 0)

    for b in range(NBUF):
        write_wait((ROUNDS - 1) * NBUF + b, b)


@jax.jit
def _run(x_flat, t_pairs, pos_table):
    mesh = plsc.VectorSubcoreMesh(core_axis_name="c", subcore_axis_name="s")
    k = functools.partial(
        pl.kernel,
        mesh=mesh,
        out_type=jax.ShapeDtypeStruct((ROWS_TOTAL, D_MODEL), jnp.float32),
        scratch_types=[
            pltpu.VMEM((ROWS_PER_W,), jnp.int32),
            pltpu.VMEM((ROWS_PER_W,), jnp.int32),
            pltpu.VMEM((SEQ_LEN, D_MODEL), jnp.float32),
            pltpu.VMEM((NBUF, CHUNK, PAIR_ROWS), jnp.float32),
            pltpu.VMEM((NBUF, CHUNK, D_MODEL), jnp.float32),
            pltpu.SemaphoreType.DMA((NBUF,)),
            pltpu.SemaphoreType.DMA((NBUF,)),
        ],
        compiler_params=pltpu.CompilerParams(
            use_tc_tiling_on_sc=True, needs_layout_passes=False),
    )(_body)
    return k(x_flat, t_pairs, pos_table)


def kernel(x, table, pos_table):
    x_flat = x.reshape(ROWS_TOTAL)
    t_pairs = table.reshape(VOCAB // 2, PAIR_ROWS)
    out = _run(x_flat, t_pairs, pos_table)
    return out.reshape(BATCH, SEQ_LEN, D_MODEL)


# speedup vs baseline: 1.1175x; 1.1175x over previous
import functools

import jax
import jax.numpy as jnp
from jax import lax
from jax.experimental import pallas as pl
from jax.experimental.pallas import tpu as pltpu
from jax.experimental.pallas import tpu_sc as plsc

VOCAB = 1000000
D_MODEL = 64
SEQ_LEN = 200
BATCH = 1024

NUM_CORES = 2
NUM_SUBCORES = 16
NW = NUM_CORES * NUM_SUBCORES

ROWS_TOTAL = BATCH * SEQ_LEN
ROWS_PER_W = ROWS_TOTAL // NW
CHUNK = 100
CHUNKS_PER_W = ROWS_PER_W // CHUNK
HALF = SEQ_LEN // CHUNK
NBUF = 4
ROUNDS = CHUNKS_PER_W // NBUF


def _body(x_hbm, table_hbm, pos_hbm, out_hbm,
          idx_v, pos_v, gbuf, wbuf, gsems, wsems):
    wid = lax.axis_index("s") * NUM_CORES + lax.axis_index("c")

    pltpu.sync_copy(x_hbm.at[wid], idx_v)
    pltpu.sync_copy(pos_hbm, pos_v)

    def gather_start(c, b):
        pltpu.make_async_copy(
            table_hbm.at[idx_v.at[c]], gbuf.at[b], gsems.at[b]).start()

    def gather_wait(c, b):
        pltpu.make_async_copy(
            table_hbm.at[idx_v.at[c]], gbuf.at[b], gsems.at[b]).wait()

    def write_start(c, b):
        pltpu.make_async_copy(
            wbuf.at[b], out_hbm.at[wid, c], wsems.at[b]).start()

    def write_wait(c, b):
        pltpu.make_async_copy(
            wbuf.at[b], out_hbm.at[wid, c], wsems.at[b]).wait()

    for b in range(NBUF):
        gather_start(b, b)

    def round_body(r, carry):
        for b in range(NBUF):
            c = r * NBUF + b
            gather_wait(c, b)

            @pl.when(r > 0)
            def _():
                write_wait(c - NBUF, b)

            par = b % HALF

            def add_row(row, carry2):
                for cc in range(D_MODEL // 16):
                    sl = pl.ds(cc * 16, 16)
                    wbuf[b, row, sl] = gbuf[b, row, sl] + pos_v[par, row, sl]
                return carry2

            lax.fori_loop(0, CHUNK, add_row, 0)

            @pl.when(r < ROUNDS - 1)
            def _():
                gather_start(c + NBUF, b)

            write_start(c, b)
        return carry

    lax.fori_loop(0, ROUNDS, round_body, 0)

    for b in range(NBUF):
        write_wait((ROUNDS - 1) * NBUF + b, b)


@jax.jit
def _run(x_r, table, pos_r):
    mesh = plsc.VectorSubcoreMesh(core_axis_name="c", subcore_axis_name="s")
    k = functools.partial(
        pl.kernel,
        mesh=mesh,
        out_type=jax.ShapeDtypeStruct((NW, CHUNKS_PER_W, CHUNK, D_MODEL), jnp.float32),
        scratch_types=[
            pltpu.VMEM((CHUNKS_PER_W, CHUNK), jnp.int32),
            pltpu.VMEM((HALF, CHUNK, D_MODEL), jnp.float32),
            pltpu.VMEM((NBUF, CHUNK, D_MODEL), jnp.float32),
            pltpu.VMEM((NBUF, CHUNK, D_MODEL), jnp.float32),
            pltpu.SemaphoreType.DMA((NBUF,)),
            pltpu.SemaphoreType.DMA((NBUF,)),
        ],
        compiler_params=pltpu.CompilerParams(use_tc_tiling_on_sc=False),
    )(_body)
    return k(x_r, table, pos_r)


def kernel(x, table, pos_table):
    x_r = x.reshape(NW, CHUNKS_PER_W, CHUNK)
    pos_r = pos_table.reshape(HALF, CHUNK, D_MODEL)
    out = _run(x_r, table, pos_r)
    return out.reshape(BATCH, SEQ_LEN, D_MODEL)
